# Initial kernel scaffold; baseline (speedup 1.0000x reference)
import jax
import jax.numpy as jnp
from jax.experimental import pallas as pl


def kernel(x, feature_table, index_table):
    raise NotImplementedError("write your pallas kernel here")


# speedup vs baseline: 100.4774x; 100.4774x over previous
import functools
import numpy as np
import jax
import jax.numpy as jnp
from jax import lax
from jax.experimental import pallas as pl
from jax.experimental.pallas import tpu as pltpu
from jax.experimental.pallas import tpu_sc as plsc

DIM = 3
NF = 4
NL = 8
TBL = 524288
MASKC = np.uint32(TBL - 1)
NPTS = 131072
RESOLUTION = 128.0
P1 = np.uint32(2654435761)
P2 = np.uint32(805459861)

NC = 2
NS = 16
NW = NC * NS
PPW = NPTS // NW
CHK = 128
NCHUNK = PPW // CHK
NGRP = CHK // 16


def _splat(v):
    return jnp.full((16,), v, jnp.int32)


def _sc_body(x_hbm, ft_hbm, it_hbm, out_hbm,
             x_buf, hidx, w_buf, fidx, itb, cob, ftb, ob, semb, semd):
    cid = lax.axis_index("c")
    sid = lax.axis_index("s")
    wid = sid * NC + cid
    iota = lax.iota(jnp.int32, 16)

    def chunk_body(k, carry):
        base = wid * PPW + k * CHK
        pltpu.sync_copy(x_hbm.at[pl.ds(base * DIM, CHK * DIM)], x_buf)

        def phase_a(g, c2):
            n0 = g * 16
            ii = iota + n0
            i3 = ii * 3
            xs = [plsc.load_gather(x_buf, [i3 + d]) * RESOLUTION
                  for d in range(DIM)]
            xi = [v.astype(jnp.int32) for v in xs]
            xf = [xs[d] - xi[d].astype(jnp.float32) for d in range(DIM)]
            t0 = [1.0 - v for v in xf]
            xu = [v.astype(jnp.uint32) for v in xi]
            m0 = [xu[0], xu[1] * P1, xu[2] * P2]
            m1 = [m0[0] + np.uint32(1), m0[1] + P1, m0[2] + P2]
            for c in range(8):
                b = [(c >> d) & 1 for d in range(DIM)]
                h = ((m1[0] if b[0] else m0[0])
                     ^ (m1[1] if b[1] else m0[1])
                     ^ (m1[2] if b[2] else m0[2]))
                hm = h & MASKC
                hidx[c, pl.ds(n0, 16)] = hm.astype(jnp.int32)
                w = ((xf[0] if b[0] else t0[0])
                     * (xf[1] if b[1] else t0[1])
                     * (xf[2] if b[2] else t0[2]))
                w_buf[c, pl.ds(n0, 16)] = w
                hp = hm * P2
                for l in range(NL):
                    cl = np.uint32((l * int(P1)) & 0xFFFFFFFF)
                    al = (hp ^ cl) & MASKC
                    fidx[c * NL + l, pl.ds(n0, 16)] = al.astype(jnp.int32)
            return c2

        lax.fori_loop(0, NGRP, phase_a, 0)

        cps_b = [pltpu.async_copy(it_hbm.at[hidx.at[j]], itb.at[j], semb)
                 for j in range(8)]
        cps_d = [pltpu.async_copy(ft_hbm.at[fidx.at[j]], ftb.at[j], semd)
                 for j in range(64)]
        for cp in cps_b:
            cp.wait()

        def phase_c(g, c2):
            n0 = g * 16
            ii = iota + n0
            for c in range(8):
                vl = [plsc.load_gather(itb, [_splat(c), ii, _splat(l)])
                      for l in range(NL)]
                mx = vl[0]
                for l in range(1, NL):
                    mx = jnp.maximum(mx, vl[l])
                ex = [jnp.exp(v - mx) for v in vl]
                s = ex[0]
                for l in range(1, NL):
                    s = s + ex[l]
                wv = w_buf[c, pl.ds(n0, 16)]
                scale = wv / s
                for l in range(NL):
                    cob[c, l, pl.ds(n0, 16)] = ex[l] * scale
            return c2

        lax.fori_loop(0, NGRP, phase_c, 0)

        for cp in cps_d:
            cp.wait()

        def phase_e(g, c2):
            n0 = g * 16
            ii = iota + n0
            acc = [jnp.zeros((16,), jnp.float32) for _ in range(NF)]
            for c in range(8):
                for l in range(NL):
                    j = c * NL + l
                    cv = cob[c, l, pl.ds(n0, 16)]
                    for f in range(NF):
                        fv = plsc.load_gather(ftb, [_splat(j), ii, _splat(f)])
                        acc[f] = acc[f] + cv * fv
            i4 = ii * NF
            for f in range(NF):
                plsc.store_scatter(ob, [i4 + f], acc[f])
            return c2

        lax.fori_loop(0, NGRP, phase_e, 0)

        pltpu.sync_copy(ob, out_hbm.at[pl.ds(base * NF, CHK * NF)])
        return carry

    lax.fori_loop(0, NCHUNK, chunk_body, 0)


@jax.jit
def _run(x, feature_table, index_table):
    mesh = plsc.VectorSubcoreMesh(core_axis_name="c", subcore_axis_name="s")
    f = pl.kernel(
        _sc_body,
        mesh=mesh,
        compiler_params=pltpu.CompilerParams(
            needs_layout_passes=False, use_tc_tiling_on_sc=False),
        out_type=jax.ShapeDtypeStruct((NPTS * NF,), jnp.float32),
        scratch_types=[
            pltpu.VMEM((CHK * DIM,), jnp.float32),
            pltpu.VMEM((8, CHK), jnp.int32),
            pltpu.VMEM((8, CHK), jnp.float32),
            pltpu.VMEM((64, CHK), jnp.int32),
            pltpu.VMEM((8, CHK, NL), jnp.float32),
            pltpu.VMEM((8, NL, CHK), jnp.float32),
            pltpu.VMEM((64, CHK, 2 * NF), jnp.float32),
            pltpu.VMEM((CHK * NF,), jnp.float32),
            pltpu.SemaphoreType.DMA,
            pltpu.SemaphoreType.DMA,
        ],
    )
    ftp = jnp.concatenate([feature_table, jnp.zeros_like(feature_table)], axis=1)
    out = f(x.reshape(-1), ftp, index_table)
    return out.reshape(NPTS, NF)


def kernel(x, feature_table, index_table):
    return _run(x, feature_table, index_table)


# speedup vs baseline: 286.4432x; 2.8508x over previous
import functools
import numpy as np
import jax
import jax.numpy as jnp
from jax import lax
from jax.experimental import pallas as pl
from jax.experimental.pallas import tpu as pltpu
from jax.experimental.pallas import tpu_sc as plsc

DIM = 3
NF = 4
NL = 8
TBL = 524288
MASKC = np.uint32(TBL - 1)
NPTS = 131072
RESOLUTION = 128.0
P1 = np.uint32(2654435761)
P2 = np.uint32(805459861)

NC = 2
NS = 16
NW = NC * NS
PPW = NPTS // NW
CHK = 128
NCHUNK = PPW // CHK
NGRP = CHK // 16

NTILE_T = TBL // 128
TPW = NTILE_T // NW
TPB = 16
NBLK = TPW // TPB


def _splat(v):
    return jnp.full((16,), v, jnp.int32)


def _transpose_body(li_hbm, lf_hbm, itl_hbm, ftl_hbm,
                    ibi, obi, ibf, obf, sem):
    cid = lax.axis_index("c")
    sid = lax.axis_index("s")
    wid = sid * NC + cid
    iota = lax.iota(jnp.int32, 16)
    t_start = wid * TPW

    def block(b, carry):
        t0 = t_start + b * TPB
        pltpu.sync_copy(li_hbm.at[pl.ds(t0 * 8, TPB * 8)], ibi)
        for tt in range(TPB):
            for g in range(8):
                row = iota + (tt * 128 + g * 16)
                for r in range(8):
                    v = ibi[tt * 8 + r, pl.ds(g * 16, 16)]
                    plsc.store_scatter(obi, [row, _splat(r)], v)
        pltpu.sync_copy(obi, itl_hbm.at[pl.ds(t0 * 128, TPB * 128)])
        pltpu.sync_copy(lf_hbm.at[pl.ds(t0 * 4, TPB * 4)], ibf)
        for tt in range(TPB):
            for g in range(8):
                row = iota + (tt * 128 + g * 16)
                for r in range(NF):
                    v = ibf[tt * 4 + r, pl.ds(g * 16, 16)]
                    plsc.store_scatter(obf, [row, _splat(r)], v)
        pltpu.sync_copy(obf, ftl_hbm.at[pl.ds(t0 * 128, TPB * 128)])
        return carry

    lax.fori_loop(0, NBLK, block, 0)


def _main_body(lx_hbm, ftl_hbm, itl_hbm, lo_hbm,
               xbuf, hidx, w_buf, fidx, itb, cob, ftb, ob, semb, semd):
    cid = lax.axis_index("c")
    sid = lax.axis_index("s")
    wid = sid * NC + cid
    iota = lax.iota(jnp.int32, 16)

    def chunk_body(k, carry):
        t = wid * NCHUNK + k
        pltpu.sync_copy(lx_hbm.at[pl.ds(t * 4, 4)], xbuf)

        def phase_a(g, c2):
            n0 = g * 16
            xs = [xbuf[d, pl.ds(n0, 16)] * RESOLUTION for d in range(DIM)]
            xi = [v.astype(jnp.int32) for v in xs]
            xf = [xs[d] - xi[d].astype(jnp.float32) for d in range(DIM)]
            t0 = [1.0 - v for v in xf]
            xu = [v.astype(jnp.uint32) for v in xi]
            m0 = [xu[0], xu[1] * P1, xu[2] * P2]
            m1 = [m0[0] + np.uint32(1), m0[1] + P1, m0[2] + P2]
            for c in range(8):
                b = [(c >> d) & 1 for d in range(DIM)]
                h = ((m1[0] if b[0] else m0[0])
                     ^ (m1[1] if b[1] else m0[1])
                     ^ (m1[2] if b[2] else m0[2]))
                hm = h & MASKC
                hidx[c, pl.ds(n0, 16)] = hm.astype(jnp.int32)
                w = ((xf[0] if b[0] else t0[0])
                     * (xf[1] if b[1] else t0[1])
                     * (xf[2] if b[2] else t0[2]))
                w_buf[c, pl.ds(n0, 16)] = w
                hp = hm * P2
                for l in range(NL):
                    cl = np.uint32((l * int(P1)) & 0xFFFFFFFF)
                    al = (hp ^ cl) & MASKC
                    fidx[c * NL + l, pl.ds(n0, 16)] = al.astype(jnp.int32)
            return c2

        lax.fori_loop(0, NGRP, phase_a, 0)

        cps_b = [pltpu.async_copy(itl_hbm.at[hidx.at[j]], itb.at[j], semb)
                 for j in range(8)]
        cps_d = [pltpu.async_copy(ftl_hbm.at[fidx.at[j]], ftb.at[j], semd)
                 for j in range(64)]
        for cp in cps_b:
            cp.wait()

        def phase_c(g, c2):
            n0 = g * 16
            ii = iota + n0
            for c in range(8):
                vl = [plsc.load_gather(itb, [_splat(c), ii, _splat(l)])
                      for l in range(NL)]
                mx = vl[0]
                for l in range(1, NL):
                    mx = jnp.maximum(mx, vl[l])
                ex = [jnp.exp(v - mx) for v in vl]
                s = ex[0]
                for l in range(1, NL):
                    s = s + ex[l]
                wv = w_buf[c, pl.ds(n0, 16)]
                scale = wv / s
                for l in range(NL):
                    cob[c, l, pl.ds(n0, 16)] = ex[l] * scale
            return c2

        lax.fori_loop(0, NGRP, phase_c, 0)

        for cp in cps_d:
            cp.wait()

        def phase_e(g, c2):
            n0 = g * 16
            ii = iota + n0
            acc = [jnp.zeros((16,), jnp.float32) for _ in range(NF)]
            for c in range(8):
                for l in range(NL):
                    j = c * NL + l
                    cv = cob[c, l, pl.ds(n0, 16)]
                    for f in range(NF):
                        fv = plsc.load_gather(ftb, [_splat(j), ii, _splat(f)])
                        acc[f] = acc[f] + cv * fv
            for f in range(NF):
                ob[f, pl.ds(n0, 16)] = acc[f]
            return c2

        lax.fori_loop(0, NGRP, phase_e, 0)

        pltpu.sync_copy(ob, lo_hbm.at[pl.ds(t * 4, 4)])
        return carry

    lax.fori_loop(0, NCHUNK, chunk_body, 0)


@jax.jit
def _run(x, feature_table, index_table):
    mesh = plsc.VectorSubcoreMesh(core_axis_name="c", subcore_axis_name="s")
    params = pltpu.CompilerParams(
        needs_layout_passes=False, use_tc_tiling_on_sc=False)

    trans = pl.kernel(
        _transpose_body,
        mesh=mesh,
        compiler_params=params,
        out_type=(jax.ShapeDtypeStruct((TBL, NL), jnp.float32),
                  jax.ShapeDtypeStruct((TBL, NL), jnp.float32)),
        scratch_types=[
            pltpu.VMEM((TPB * 8, 128), jnp.float32),
            pltpu.VMEM((TPB * 128, NL), jnp.float32),
            pltpu.VMEM((TPB * 4, 128), jnp.float32),
            pltpu.VMEM((TPB * 128, NL), jnp.float32),
            pltpu.SemaphoreType.DMA,
        ],
    )

    main = pl.kernel(
        _main_body,
        mesh=mesh,
        compiler_params=params,
        out_type=jax.ShapeDtypeStruct((NPTS // 128 * 4, 128), jnp.float32),
        scratch_types=[
            pltpu.VMEM((4, 128), jnp.float32),
            pltpu.VMEM((8, CHK), jnp.int32),
            pltpu.VMEM((8, CHK), jnp.float32),
            pltpu.VMEM((64, CHK), jnp.int32),
            pltpu.VMEM((8, CHK, NL), jnp.float32),
            pltpu.VMEM((8, NL, CHK), jnp.float32),
            pltpu.VMEM((64, CHK, NL), jnp.float32),
            pltpu.VMEM((4, 128), jnp.float32),
            pltpu.SemaphoreType.DMA,
            pltpu.SemaphoreType.DMA,
        ],
    )

    li = index_table.T.reshape(NL, NTILE_T, 128).transpose(1, 0, 2)
    li = li.reshape(NTILE_T * NL, 128)
    lf = feature_table.T.reshape(NF, NTILE_T, 128).transpose(1, 0, 2)
    lf = lf.reshape(NTILE_T * NF, 128)
    xp = jnp.pad(x, ((0, 0), (0, 1)))
    lx = xp.T.reshape(4, NPTS // 128, 128).transpose(1, 0, 2)
    lx = lx.reshape(NPTS // 128 * 4, 128)

    itl, ftl = trans(li, lf)
    lo = main(lx, ftl, itl)

    out = lo.reshape(NPTS // 128, 4, 128).transpose(1, 0, 2)
    return out.reshape(4, NPTS).T


def kernel(x, feature_table, index_table):
    return _run(x, feature_table, index_table)


# speedup vs baseline: 361.0360x; 1.2604x over previous
import functools
import numpy as np
import jax
import jax.numpy as jnp
from jax import lax
from jax.experimental import pallas as pl
from jax.experimental.pallas import tpu as pltpu
from jax.experimental.pallas import tpu_sc as plsc

DIM = 3
NF = 4
NL = 8
TBL = 524288
MASKC = np.uint32(TBL - 1)
NPTS = 131072
RESOLUTION = 128.0
P1 = np.uint32(2654435761)
P2 = np.uint32(805459861)

NC = 2
NS = 16
NW = NC * NS

NTILE_T = TBL // 128
TPW = NTILE_T // NW

TPB = 16
NBLK = TPW // TPB

PT_B = 4
PNB = TPW // PT_B

CHK = 128
NCHUNK = NPTS // NW // CHK
NGRP = CHK // 16


def _splat(v):
    return jnp.full((16,), v, jnp.int32)


def _detile_ft_body(lf_hbm, ftl_hbm, ibf, obf, sem):
    cid = lax.axis_index("c")
    sid = lax.axis_index("s")
    wid = sid * NC + cid
    iota = lax.iota(jnp.int32, 16)
    t_start = wid * TPW

    def block(b, carry):
        t0 = t_start + b * TPB
        pltpu.sync_copy(lf_hbm.at[pl.ds(t0 * NF, TPB * NF)], ibf)
        for tt in range(TPB):
            for g in range(8):
                row = iota + (tt * 128 + g * 16)
                for r in range(NF):
                    v = ibf[tt * NF + r, pl.ds(g * 16, 16)]
                    plsc.store_scatter(obf, [row, _splat(r)], v)
        pltpu.sync_copy(obf, ftl_hbm.at[pl.ds(t0 * 128, TPB * 128)])
        return carry

    lax.fori_loop(0, NBLK, block, 0)


def _precompute_body(li_hbm, ftl_hbm, f_hbm, itbuf, fidx, ftb, cob, fbuf, sem):
    cid = lax.axis_index("c")
    sid = lax.axis_index("s")
    wid = sid * NC + cid
    iota = lax.iota(jnp.int32, 16)
    t_start = wid * TPW

    def block(b, carry):
        t0 = t_start + b * PT_B
        h0 = t0 * 128
        pltpu.sync_copy(li_hbm.at[pl.ds(t0 * NL, PT_B * NL)], itbuf)

        for tt in range(PT_B):
            for g in range(8):
                hv = (iota + (h0 + tt * 128 + g * 16)).astype(jnp.uint32)
                hp = hv * P2
                for l in range(NL):
                    cl = np.uint32((l * int(P1)) & 0xFFFFFFFF)
                    al = (hp ^ cl) & MASKC
                    fidx[tt * NL + l, pl.ds(g * 16, 16)] = al.astype(jnp.int32)

        cps = [pltpu.async_copy(ftl_hbm.at[fidx.at[s]], ftb.at[s], sem)
               for s in range(PT_B * NL)]

        for tt in range(PT_B):
            for g in range(8):
                vl = [itbuf[tt * NL + l, pl.ds(g * 16, 16)] for l in range(NL)]
                mx = vl[0]
                for l in range(1, NL):
                    mx = jnp.maximum(mx, vl[l])
                ex = [jnp.exp(v - mx) for v in vl]
                s = ex[0]
                for l in range(1, NL):
                    s = s + ex[l]
                inv = 1.0 / s
                for l in range(NL):
                    cob[tt * NL + l, pl.ds(g * 16, 16)] = ex[l] * inv

        for cp in cps:
            cp.wait()

        for tt in range(PT_B):
            for g in range(8):
                ii = iota + g * 16
                row = iota + (tt * 128 + g * 16)
                acc = [jnp.zeros((16,), jnp.float32) for _ in range(NF)]
                for l in range(NL):
                    cv = cob[tt * NL + l, pl.ds(g * 16, 16)]
                    for f in range(NF):
                        fv = plsc.load_gather(
                            ftb, [_splat(tt * NL + l), ii, _splat(f)])
                        acc[f] = acc[f] + cv * fv
                for f in range(NF):
                    plsc.store_scatter(fbuf, [row, _splat(f)], acc[f])

        pltpu.sync_copy(fbuf, f_hbm.at[pl.ds(h0, PT_B * 128)])
        return carry

    lax.fori_loop(0, PNB, block, 0)


def _final_body(lx_hbm, f_hbm, lo_hbm, xbuf, hidx, w_buf, fb, ob, sem):
    cid = lax.axis_index("c")
    sid = lax.axis_index("s")
    wid = sid * NC + cid
    iota = lax.iota(jnp.int32, 16)

    def chunk_body(k, carry):
        t = wid * NCHUNK + k
        pltpu.sync_copy(lx_hbm.at[pl.ds(t * 4, 4)], xbuf)

        def phase_a(g, c2):
            n0 = g * 16
            xs = [xbuf[d, pl.ds(n0, 16)] * RESOLUTION for d in range(DIM)]
            xi = [v.astype(jnp.int32) for v in xs]
            xf = [xs[d] - xi[d].astype(jnp.float32) for d in range(DIM)]
            t0 = [1.0 - v for v in xf]
            xu = [v.astype(jnp.uint32) for v in xi]
            m0 = [xu[0], xu[1] * P1, xu[2] * P2]
            m1 = [m0[0] + np.uint32(1), m0[1] + P1, m0[2] + P2]
            for c in range(8):
                b = [(c >> d) & 1 for d in range(DIM)]
                h = ((m1[0] if b[0] else m0[0])
                     ^ (m1[1] if b[1] else m0[1])
                     ^ (m1[2] if b[2] else m0[2]))
                hidx[c, pl.ds(n0, 16)] = (h & MASKC).astype(jnp.int32)
                w = ((xf[0] if b[0] else t0[0])
                     * (xf[1] if b[1] else t0[1])
                     * (xf[2] if b[2] else t0[2]))
                w_buf[c, pl.ds(n0, 16)] = w
            return c2

        lax.fori_loop(0, NGRP, phase_a, 0)

        cps = [pltpu.async_copy(f_hbm.at[hidx.at[j]], fb.at[j], sem)
               for j in range(8)]
        for cp in cps:
            cp.wait()

        def reduce_g(g, c2):
            n0 = g * 16
            ii = iota + n0
            acc = [jnp.zeros((16,), jnp.float32) for _ in range(NF)]
            for c in range(8):
                wv = w_buf[c, pl.ds(n0, 16)]
                for f in range(NF):
                    fv = plsc.load_gather(fb, [_splat(c), ii, _splat(f)])
                    acc[f] = acc[f] + wv * fv
            for f in range(NF):
                ob[f, pl.ds(n0, 16)] = acc[f]
            return c2

        lax.fori_loop(0, NGRP, reduce_g, 0)

        pltpu.sync_copy(ob, lo_hbm.at[pl.ds(t * 4, 4)])
        return carry

    lax.fori_loop(0, NCHUNK, chunk_body, 0)


@jax.jit
def _run(x, feature_table, index_table):
    mesh = plsc.VectorSubcoreMesh(core_axis_name="c", subcore_axis_name="s")
    params = pltpu.CompilerParams(
        needs_layout_passes=False, use_tc_tiling_on_sc=False)

    detile_ft = pl.kernel(
        _detile_ft_body,
        mesh=mesh,
        compiler_params=params,
        out_type=jax.ShapeDtypeStruct((TBL, NL), jnp.float32),
        scratch_types=[
            pltpu.VMEM((TPB * NF, 128), jnp.float32),
            pltpu.VMEM((TPB * 128, NL), jnp.float32),
            pltpu.SemaphoreType.DMA,
        ],
    )

    precompute = pl.kernel(
        _precompute_body,
        mesh=mesh,
        compiler_params=params,
        out_type=jax.ShapeDtypeStruct((TBL, NL), jnp.float32),
        scratch_types=[
            pltpu.VMEM((PT_B * NL, 128), jnp.float32),
            pltpu.VMEM((PT_B * NL, 128), jnp.int32),
            pltpu.VMEM((PT_B * NL, 128, NL), jnp.float32),
            pltpu.VMEM((PT_B * NL, 128), jnp.float32),
            pltpu.VMEM((PT_B * 128, NL), jnp.float32),
            pltpu.SemaphoreType.DMA,
        ],
    )

    final = pl.kernel(
        _final_body,
        mesh=mesh,
        compiler_params=params,
        out_type=jax.ShapeDtypeStruct((NPTS // 128 * 4, 128), jnp.float32),
        scratch_types=[
            pltpu.VMEM((4, 128), jnp.float32),
            pltpu.VMEM((8, CHK), jnp.int32),
            pltpu.VMEM((8, CHK), jnp.float32),
            pltpu.VMEM((8, CHK, NL), jnp.float32),
            pltpu.VMEM((4, 128), jnp.float32),
            pltpu.SemaphoreType.DMA,
        ],
    )

    li = index_table.T.reshape(NL, NTILE_T, 128).transpose(1, 0, 2)
    li = li.reshape(NTILE_T * NL, 128)
    lf = feature_table.T.reshape(NF, NTILE_T, 128).transpose(1, 0, 2)
    lf = lf.reshape(NTILE_T * NF, 128)
    xp = jnp.pad(x, ((0, 0), (0, 1)))
    lx = xp.T.reshape(4, NPTS // 128, 128).transpose(1, 0, 2)
    lx = lx.reshape(NPTS // 128 * 4, 128)

    ftl = detile_ft(lf)
    fq = precompute(li, ftl)
    lo = final(lx, fq)

    out = lo.reshape(NPTS // 128, 4, 128).transpose(1, 0, 2)
    return out.reshape(4, NPTS).T


def kernel(x, feature_table, index_table):
    return _run(x, feature_table, index_table)


# speedup vs baseline: 460.1707x; 1.2746x over previous
import functools
import numpy as np
import jax
import jax.numpy as jnp
from jax import lax
from jax.experimental import pallas as pl
from jax.experimental.pallas import tpu as pltpu
from jax.experimental.pallas import tpu_sc as plsc

DIM = 3
NF = 4
NL = 8
TBL = 524288
MASKC = np.uint32(TBL - 1)
NPTS = 131072
RESOLUTION = 128.0
P1 = np.uint32(2654435761)
P2 = np.uint32(805459861)

NC = 2
NS = 16
NW = NC * NS

NTILE_T = TBL // 128
TPW = NTILE_T // NW

TPB = 16
NBLK = TPW // TPB

PT_B = 4
PNB = TPW // PT_B

CHK = 128
NCHUNK = NPTS // NW // CHK
NGRP = CHK // 16


def _splat(v):
    return jnp.full((16,), v, jnp.int32)


def _detile_ft_body(lf_hbm, ftl_hbm, ibf, obf, sem):
    cid = lax.axis_index("c")
    sid = lax.axis_index("s")
    wid = sid * NC + cid
    iota = lax.iota(jnp.int32, 16)
    t_start = wid * TPW

    def block(b, carry):
        t0 = t_start + b * TPB
        pltpu.sync_copy(lf_hbm.at[pl.ds(t0 * NF, TPB * NF)], ibf)
        for tt in range(TPB):
            for g in range(8):
                row = iota + (tt * 128 + g * 16)
                for r in range(NF):
                    v = ibf[tt * NF + r, pl.ds(g * 16, 16)]
                    plsc.store_scatter(obf, [row, _splat(r)], v)
        pltpu.sync_copy(obf, ftl_hbm.at[pl.ds(t0 * 128, TPB * 128)])
        return carry

    lax.fori_loop(0, NBLK, block, 0)


NROW_B = PT_B * NL


def _precompute_body(li_hbm, ftl_hbm, f_hbm, itbuf, fidx, ftb, cob, fbuf,
                     sem_li0, sem_li1, sem_ft0, sem_ft1):
    cid = lax.axis_index("c")
    sid = lax.axis_index("s")
    wid = sid * NC + cid
    iota = lax.iota(jnp.int32, 16)
    t_start = wid * TPW

    def stage(b, o, sem_ft):
        h0 = (t_start + b * PT_B) * 128
        for tt in range(PT_B):
            for g in range(8):
                hv = (iota + (h0 + tt * 128 + g * 16)).astype(jnp.uint32)
                hp = hv * P2
                for l in range(NL):
                    cl = np.uint32((l * int(P1)) & 0xFFFFFFFF)
                    al = (hp ^ cl) & MASKC
                    fidx[o + tt * NL + l, pl.ds(g * 16, 16)] = (
                        al.astype(jnp.int32))
        for s in range(NROW_B):
            pltpu.async_copy(ftl_hbm.at[fidx.at[o + s]], ftb.at[o + s],
                             sem_ft)

    def load_li(b, o, sem_li):
        t0 = t_start + b * PT_B
        pltpu.async_copy(li_hbm.at[pl.ds(t0 * NL, PT_B * NL)],
                         itbuf.at[pl.ds(o, PT_B * NL)], sem_li)

    def consume(b, o, sem_li, sem_ft):
        h0 = (t_start + b * PT_B) * 128
        pltpu.make_async_copy(
            li_hbm.at[pl.ds(t_start * NL, PT_B * NL)],
            itbuf.at[pl.ds(o, PT_B * NL)], sem_li).wait()
        for tt in range(PT_B):
            for g in range(8):
                vl = [itbuf[o + tt * NL + l, pl.ds(g * 16, 16)]
                      for l in range(NL)]
                mx = vl[0]
                for l in range(1, NL):
                    mx = jnp.maximum(mx, vl[l])
                ex = [jnp.exp(v - mx) for v in vl]
                s = ex[0]
                for l in range(1, NL):
                    s = s + ex[l]
                inv = 1.0 / s
                for l in range(NL):
                    cob[tt * NL + l, pl.ds(g * 16, 16)] = ex[l] * inv
        for s in range(NROW_B):
            pltpu.make_async_copy(ftl_hbm.at[fidx.at[o + s]],
                                  ftb.at[o + s], sem_ft).wait()
        for tt in range(PT_B):
            for g in range(8):
                ii = iota + g * 16
                row = iota + (tt * 128 + g * 16)
                acc = [jnp.zeros((16,), jnp.float32) for _ in range(NF)]
                for l in range(NL):
                    cv = cob[tt * NL + l, pl.ds(g * 16, 16)]
                    for f in range(NF):
                        fv = plsc.load_gather(
                            ftb, [_splat(tt * NL + l) + o, ii, _splat(f)])
                        acc[f] = acc[f] + cv * fv
                for f in range(NF):
                    plsc.store_scatter(fbuf, [row, _splat(f)], acc[f])
        pltpu.sync_copy(fbuf, f_hbm.at[pl.ds(h0, PT_B * 128)])

    load_li(0, 0, sem_li0)
    stage(0, 0, sem_ft0)

    def two_blocks(i, carry):
        b0 = i * 2
        load_li(b0 + 1, NROW_B, sem_li1)
        stage(b0 + 1, NROW_B, sem_ft1)
        consume(b0, 0, sem_li0, sem_ft0)

        @pl.when(b0 + 2 < PNB)
        def _():
            load_li(b0 + 2, 0, sem_li0)
            stage(b0 + 2, 0, sem_ft0)

        consume(b0 + 1, NROW_B, sem_li1, sem_ft1)
        return carry

    lax.fori_loop(0, PNB // 2, two_blocks, 0)


def _final_body(lx_hbm, f_hbm, lo_hbm, xbuf, hidx, w_buf, fb, ob, sem):
    cid = lax.axis_index("c")
    sid = lax.axis_index("s")
    wid = sid * NC + cid
    iota = lax.iota(jnp.int32, 16)

    def chunk_body(k, carry):
        t = wid * NCHUNK + k
        pltpu.sync_copy(lx_hbm.at[pl.ds(t * 4, 4)], xbuf)

        def phase_a(g, c2):
            n0 = g * 16
            xs = [xbuf[d, pl.ds(n0, 16)] * RESOLUTION for d in range(DIM)]
            xi = [v.astype(jnp.int32) for v in xs]
            xf = [xs[d] - xi[d].astype(jnp.float32) for d in range(DIM)]
            t0 = [1.0 - v for v in xf]
            xu = [v.astype(jnp.uint32) for v in xi]
            m0 = [xu[0], xu[1] * P1, xu[2] * P2]
            m1 = [m0[0] + np.uint32(1), m0[1] + P1, m0[2] + P2]
            for c in range(8):
                b = [(c >> d) & 1 for d in range(DIM)]
                h = ((m1[0] if b[0] else m0[0])
                     ^ (m1[1] if b[1] else m0[1])
                     ^ (m1[2] if b[2] else m0[2]))
                hidx[c, pl.ds(n0, 16)] = (h & MASKC).astype(jnp.int32)
                w = ((xf[0] if b[0] else t0[0])
                     * (xf[1] if b[1] else t0[1])
                     * (xf[2] if b[2] else t0[2]))
                w_buf[c, pl.ds(n0, 16)] = w
            return c2

        lax.fori_loop(0, NGRP, phase_a, 0)

        cps = [pltpu.async_copy(f_hbm.at[hidx.at[j]], fb.at[j], sem)
               for j in range(8)]
        for cp in cps:
            cp.wait()

        def reduce_g(g, c2):
            n0 = g * 16
            ii = iota + n0
            acc = [jnp.zeros((16,), jnp.float32) for _ in range(NF)]
            for c in range(8):
                wv = w_buf[c, pl.ds(n0, 16)]
                for f in range(NF):
                    fv = plsc.load_gather(fb, [_splat(c), ii, _splat(f)])
                    acc[f] = acc[f] + wv * fv
            for f in range(NF):
                ob[f, pl.ds(n0, 16)] = acc[f]
            return c2

        lax.fori_loop(0, NGRP, reduce_g, 0)

        pltpu.sync_copy(ob, lo_hbm.at[pl.ds(t * 4, 4)])
        return carry

    lax.fori_loop(0, NCHUNK, chunk_body, 0)


@jax.jit
def _run(x, feature_table, index_table):
    mesh = plsc.VectorSubcoreMesh(core_axis_name="c", subcore_axis_name="s")
    params = pltpu.CompilerParams(
        needs_layout_passes=False, use_tc_tiling_on_sc=False)

    detile_ft = pl.kernel(
        _detile_ft_body,
        mesh=mesh,
        compiler_params=params,
        out_type=jax.ShapeDtypeStruct((TBL, NL), jnp.float32),
        scratch_types=[
            pltpu.VMEM((TPB * NF, 128), jnp.float32),
            pltpu.VMEM((TPB * 128, NL), jnp.float32),
            pltpu.SemaphoreType.DMA,
        ],
    )

    precompute = pl.kernel(
        _precompute_body,
        mesh=mesh,
        compiler_params=params,
        out_type=jax.ShapeDtypeStruct((TBL, NL), jnp.float32),
        scratch_types=[
            pltpu.VMEM((2 * NROW_B, 128), jnp.float32),
            pltpu.VMEM((2 * NROW_B, 128), jnp.int32),
            pltpu.VMEM((2 * NROW_B, 128, NL), jnp.float32),
            pltpu.VMEM((NROW_B, 128), jnp.float32),
            pltpu.VMEM((PT_B * 128, NL), jnp.float32),
            pltpu.SemaphoreType.DMA,
            pltpu.SemaphoreType.DMA,
            pltpu.SemaphoreType.DMA,
            pltpu.SemaphoreType.DMA,
        ],
    )

    final = pl.kernel(
        _final_body,
        mesh=mesh,
        compiler_params=params,
        out_type=jax.ShapeDtypeStruct((NPTS // 128 * 4, 128), jnp.float32),
        scratch_types=[
            pltpu.VMEM((4, 128), jnp.float32),
            pltpu.VMEM((8, CHK), jnp.int32),
            pltpu.VMEM((8, CHK), jnp.float32),
            pltpu.VMEM((8, CHK, NL), jnp.float32),
            pltpu.VMEM((4, 128), jnp.float32),
            pltpu.SemaphoreType.DMA,
        ],
    )

    li = index_table.T.reshape(NL, NTILE_T, 128).transpose(1, 0, 2)
    li = li.reshape(NTILE_T * NL, 128)
    lf = feature_table.T.reshape(NF, NTILE_T, 128).transpose(1, 0, 2)
    lf = lf.reshape(NTILE_T * NF, 128)
    xp = jnp.pad(x, ((0, 0), (0, 1)))
    lx = xp.T.reshape(4, NPTS // 128, 128).transpose(1, 0, 2)
    lx = lx.reshape(NPTS // 128 * 4, 128)

    ftl = detile_ft(lf)
    fq = precompute(li, ftl)
    lo = final(lx, fq)

    out = lo.reshape(NPTS // 128, 4, 128).transpose(1, 0, 2)
    return out.reshape(4, NPTS).T


def kernel(x, feature_table, index_table):
    return _run(x, feature_table, index_table)


# speedup vs baseline: 521.9985x; 1.1344x over previous
import functools
import numpy as np
import jax
import jax.numpy as jnp
from jax import lax
from jax.experimental import pallas as pl
from jax.experimental.pallas import tpu as pltpu
from jax.experimental.pallas import tpu_sc as plsc

DIM = 3
NF = 4
NL = 8
TBL = 524288
MASKC = np.uint32(TBL - 1)
NPTS = 131072
RESOLUTION = 128.0
P1 = np.uint32(2654435761)
P2 = np.uint32(805459861)

NC = 2
NS = 16
NW = NC * NS

NTILE_T = TBL // 128
TPW = NTILE_T // NW

TPB = 16
NBLK = TPW // TPB

PT_B = 4
PNB = TPW // PT_B

CHK = 128
NCHUNK = NPTS // NW // CHK
NGRP = CHK // 16


def _splat(v):
    return jnp.full((16,), v, jnp.int32)


def _detile_ft_body(lf_hbm, ftl_hbm, ibf, obf, sem):
    cid = lax.axis_index("c")
    sid = lax.axis_index("s")
    wid = sid * NC + cid
    iota = lax.iota(jnp.int32, 16)
    t_start = wid * TPW

    def block(b, carry):
        t0 = t_start + b * TPB
        pltpu.sync_copy(lf_hbm.at[pl.ds(t0 * NF, TPB * NF)], ibf)
        for tt in range(TPB):
            for g in range(8):
                row = iota + (tt * 128 + g * 16)
                for r in range(NF):
                    v = ibf[tt * NF + r, pl.ds(g * 16, 16)]
                    plsc.store_scatter(obf, [row, _splat(r)], v)
        pltpu.sync_copy(obf, ftl_hbm.at[pl.ds(t0 * 128, TPB * 128)])
        return carry

    lax.fori_loop(0, NBLK, block, 0)


NROW_B = PT_B * NL


def _precompute_body(li_hbm, ftl_hbm, f_hbm, itbuf, fidx, ftb, cob, fbuf,
                     sem_li0, sem_li1, sem_ft0, sem_ft1):
    cid = lax.axis_index("c")
    sid = lax.axis_index("s")
    wid = sid * NC + cid
    iota = lax.iota(jnp.int32, 16)
    t_start = wid * TPW

    def stage(b, o, sem_ft):
        h0 = (t_start + b * PT_B) * 128
        for tt in range(PT_B):
            for g in range(8):
                hv = (iota + (h0 + tt * 128 + g * 16)).astype(jnp.uint32)
                hp = hv * P2
                for l in range(NL):
                    cl = np.uint32((l * int(P1)) & 0xFFFFFFFF)
                    al = (hp ^ cl) & MASKC
                    fidx[o + tt * NL + l, pl.ds(g * 16, 16)] = (
                        al.astype(jnp.int32))
        for s in range(NROW_B):
            pltpu.async_copy(ftl_hbm.at[fidx.at[o + s]], ftb.at[o + s],
                             sem_ft)

    def load_li(b, o, sem_li):
        t0 = t_start + b * PT_B
        pltpu.async_copy(li_hbm.at[pl.ds(t0 * NL, PT_B * NL)],
                         itbuf.at[pl.ds(o, PT_B * NL)], sem_li)

    def consume(b, o, sem_li, sem_ft):
        h0 = (t_start + b * PT_B) * 128
        pltpu.make_async_copy(
            li_hbm.at[pl.ds(t_start * NL, PT_B * NL)],
            itbuf.at[pl.ds(o, PT_B * NL)], sem_li).wait()
        for tt in range(PT_B):
            for g in range(8):
                vl = [itbuf[o + tt * NL + l, pl.ds(g * 16, 16)]
                      for l in range(NL)]
                mx = vl[0]
                for l in range(1, NL):
                    mx = jnp.maximum(mx, vl[l])
                ex = [jnp.exp(v - mx) for v in vl]
                s = ex[0]
                for l in range(1, NL):
                    s = s + ex[l]
                inv = 1.0 / s
                for l in range(NL):
                    cob[tt * NL + l, pl.ds(g * 16, 16)] = ex[l] * inv
        for s in range(NROW_B):
            pltpu.make_async_copy(ftl_hbm.at[fidx.at[o + s]],
                                  ftb.at[o + s], sem_ft).wait()
        for tt in range(PT_B):
            for g in range(8):
                ii = iota + g * 16
                row = iota + (tt * 128 + g * 16)
                acc = [jnp.zeros((16,), jnp.float32) for _ in range(NF)]
                for l in range(NL):
                    cv = cob[tt * NL + l, pl.ds(g * 16, 16)]
                    for f in range(NF):
                        fv = plsc.load_gather(
                            ftb, [_splat(tt * NL + l) + o, ii, _splat(f)])
                        acc[f] = acc[f] + cv * fv
                for f in range(NF):
                    plsc.store_scatter(fbuf, [row, _splat(f)], acc[f])
        pltpu.sync_copy(fbuf, f_hbm.at[pl.ds(h0, PT_B * 128)])

    load_li(0, 0, sem_li0)
    stage(0, 0, sem_ft0)

    def two_blocks(i, carry):
        b0 = i * 2
        load_li(b0 + 1, NROW_B, sem_li1)
        stage(b0 + 1, NROW_B, sem_ft1)
        consume(b0, 0, sem_li0, sem_ft0)

        @pl.when(b0 + 2 < PNB)
        def _():
            load_li(b0 + 2, 0, sem_li0)
            stage(b0 + 2, 0, sem_ft0)

        consume(b0 + 1, NROW_B, sem_li1, sem_ft1)
        return carry

    lax.fori_loop(0, PNB // 2, two_blocks, 0)


def _final_body(lx_hbm, f_hbm, lo_hbm, xbuf, hidx, w_buf, fb, ob,
                semx0, semx1, semf0, semf1):
    cid = lax.axis_index("c")
    sid = lax.axis_index("s")
    wid = sid * NC + cid
    iota = lax.iota(jnp.int32, 16)

    def load_x(k, o, semx):
        t = wid * NCHUNK + k
        pltpu.async_copy(lx_hbm.at[pl.ds(t * 4, 4)],
                         xbuf.at[pl.ds(o * 4, 4)], semx)

    def hash_fire(k, o, semx, semf):
        pltpu.make_async_copy(lx_hbm.at[pl.ds(wid * 4, 4)],
                              xbuf.at[pl.ds(o * 4, 4)], semx).wait()

        def phase_a(g, c2):
            n0 = g * 16
            xs = [xbuf[o * 4 + d, pl.ds(n0, 16)] * RESOLUTION
                  for d in range(DIM)]
            xi = [v.astype(jnp.int32) for v in xs]
            xf = [xs[d] - xi[d].astype(jnp.float32) for d in range(DIM)]
            t0 = [1.0 - v for v in xf]
            xu = [v.astype(jnp.uint32) for v in xi]
            m0 = [xu[0], xu[1] * P1, xu[2] * P2]
            m1 = [m0[0] + np.uint32(1), m0[1] + P1, m0[2] + P2]
            for c in range(8):
                b = [(c >> d) & 1 for d in range(DIM)]
                h = ((m1[0] if b[0] else m0[0])
                     ^ (m1[1] if b[1] else m0[1])
                     ^ (m1[2] if b[2] else m0[2]))
                hidx[o * 8 + c, pl.ds(n0, 16)] = (h & MASKC).astype(jnp.int32)
                w = ((xf[0] if b[0] else t0[0])
                     * (xf[1] if b[1] else t0[1])
                     * (xf[2] if b[2] else t0[2]))
                w_buf[o * 8 + c, pl.ds(n0, 16)] = w
            return c2

        lax.fori_loop(0, NGRP, phase_a, 0)
        for j in range(8):
            pltpu.async_copy(f_hbm.at[hidx.at[o * 8 + j]],
                             fb.at[o * 8 + j], semf)

    def consume(k, o, semf):
        t = wid * NCHUNK + k
        for j in range(8):
            pltpu.make_async_copy(f_hbm.at[hidx.at[o * 8 + j]],
                                  fb.at[o * 8 + j], semf).wait()

        def reduce_g(g, c2):
            n0 = g * 16
            ii = iota + n0
            acc = [jnp.zeros((16,), jnp.float32) for _ in range(NF)]
            for c in range(8):
                wv = w_buf[o * 8 + c, pl.ds(n0, 16)]
                for f in range(NF):
                    fv = plsc.load_gather(fb, [_splat(o * 8 + c), ii,
                                               _splat(f)])
                    acc[f] = acc[f] + wv * fv
            for f in range(NF):
                ob[f, pl.ds(n0, 16)] = acc[f]
            return c2

        lax.fori_loop(0, NGRP, reduce_g, 0)
        pltpu.sync_copy(ob, lo_hbm.at[pl.ds(t * 4, 4)])

    load_x(0, 0, semx0)
    hash_fire(0, 0, semx0, semf0)

    def two_chunks(i, carry):
        k0 = i * 2
        load_x(k0 + 1, 1, semx1)
        hash_fire(k0 + 1, 1, semx1, semf1)
        consume(k0, 0, semf0)

        @pl.when(k0 + 2 < NCHUNK)
        def _():
            load_x(k0 + 2, 0, semx0)
            hash_fire(k0 + 2, 0, semx0, semf0)

        consume(k0 + 1, 1, semf1)
        return carry

    lax.fori_loop(0, NCHUNK // 2, two_chunks, 0)


@jax.jit
def _run(x, feature_table, index_table):
    mesh = plsc.VectorSubcoreMesh(core_axis_name="c", subcore_axis_name="s")
    params = pltpu.CompilerParams(
        needs_layout_passes=False, use_tc_tiling_on_sc=False)

    detile_ft = pl.kernel(
        _detile_ft_body,
        mesh=mesh,
        compiler_params=params,
        out_type=jax.ShapeDtypeStruct((TBL, NL), jnp.float32),
        scratch_types=[
            pltpu.VMEM((TPB * NF, 128), jnp.float32),
            pltpu.VMEM((TPB * 128, NL), jnp.float32),
            pltpu.SemaphoreType.DMA,
        ],
    )

    precompute = pl.kernel(
        _precompute_body,
        mesh=mesh,
        compiler_params=params,
        out_type=jax.ShapeDtypeStruct((TBL, NL), jnp.float32),
        scratch_types=[
            pltpu.VMEM((2 * NROW_B, 128), jnp.float32),
            pltpu.VMEM((2 * NROW_B, 128), jnp.int32),
            pltpu.VMEM((2 * NROW_B, 128, NL), jnp.float32),
            pltpu.VMEM((NROW_B, 128), jnp.float32),
            pltpu.VMEM((PT_B * 128, NL), jnp.float32),
            pltpu.SemaphoreType.DMA,
            pltpu.SemaphoreType.DMA,
            pltpu.SemaphoreType.DMA,
            pltpu.SemaphoreType.DMA,
        ],
    )

    final = pl.kernel(
        _final_body,
        mesh=mesh,
        compiler_params=params,
        out_type=jax.ShapeDtypeStruct((NPTS // 128 * 4, 128), jnp.float32),
        scratch_types=[
            pltpu.VMEM((2 * 4, 128), jnp.float32),
            pltpu.VMEM((2 * 8, CHK), jnp.int32),
            pltpu.VMEM((2 * 8, CHK), jnp.float32),
            pltpu.VMEM((2 * 8, CHK, NL), jnp.float32),
            pltpu.VMEM((4, 128), jnp.float32),
            pltpu.SemaphoreType.DMA,
            pltpu.SemaphoreType.DMA,
            pltpu.SemaphoreType.DMA,
            pltpu.SemaphoreType.DMA,
        ],
    )

    li = index_table.T.reshape(NL, NTILE_T, 128).transpose(1, 0, 2)
    li = li.reshape(NTILE_T * NL, 128)
    lf = feature_table.T.reshape(NF, NTILE_T, 128).transpose(1, 0, 2)
    lf = lf.reshape(NTILE_T * NF, 128)
    xp = jnp.pad(x, ((0, 0), (0, 1)))
    lx = xp.T.reshape(4, NPTS // 128, 128).transpose(1, 0, 2)
    lx = lx.reshape(NPTS // 128 * 4, 128)

    ftl = detile_ft(lf)
    fq = precompute(li, ftl)
    lo = final(lx, fq)

    out = lo.reshape(NPTS // 128, 4, 128).transpose(1, 0, 2)
    return out.reshape(4, NPTS).T


def kernel(x, feature_table, index_table):
    return _run(x, feature_table, index_table)


# speedup vs baseline: 525.9050x; 1.0075x over previous
import numpy as np
import jax
import jax.numpy as jnp
from jax import lax
from jax.experimental import pallas as pl
from jax.experimental.pallas import tpu as pltpu
from jax.experimental.pallas import tpu_sc as plsc

DIM = 3
NF = 4
NL = 8
TBL = 524288
MASKC = np.uint32(TBL - 1)
NPTS = 131072
RESOLUTION = 128.0
P1 = np.uint32(2654435761)
P2 = np.uint32(805459861)

NC = 2
NS = 16
NW = NC * NS

NTILE_T = TBL // 128
TPW = NTILE_T // NW

TPB = 16
NBLK = TPW // TPB

PT_B = 4
PNB = TPW // PT_B

CHK = 128
NCHUNK = NPTS // NW // CHK
NGRP = CHK // 16


def _splat(v):
    return jnp.full((16,), v, jnp.int32)


def _detile_ft_body(lf_hbm, ftl_hbm, ibf, obf, sem):
    cid = lax.axis_index("c")
    sid = lax.axis_index("s")
    wid = sid * NC + cid
    iota = lax.iota(jnp.int32, 16)
    t_start = wid * TPW

    def block(b, carry):
        t0 = t_start + b * TPB
        pltpu.sync_copy(lf_hbm.at[pl.ds(t0 * NF, TPB * NF)], ibf)
        for tt in range(TPB):
            for g in range(8):
                row = iota + (tt * 128 + g * 16)
                for r in range(NF):
                    v = ibf[tt * NF + r, pl.ds(g * 16, 16)]
                    plsc.store_scatter(obf, [row, _splat(r)], v)
        pltpu.sync_copy(obf, ftl_hbm.at[pl.ds(t0 * 128, TPB * 128)])
        return carry

    lax.fori_loop(0, NBLK, block, 0)


NROW_B = PT_B * NL


def _precompute_body(li_hbm, ftl_hbm, f_hbm, itbuf, fidx, ftb, cob, fbuf,
                     sem_li0, sem_li1, sem_ft0, sem_ft1):
    cid = lax.axis_index("c")
    sid = lax.axis_index("s")
    wid = sid * NC + cid
    iota = lax.iota(jnp.int32, 16)
    t_start = wid * TPW

    def stage(b, o, sem_ft):
        h0 = (t_start + b * PT_B) * 128
        for tt in range(PT_B):
            for g in range(8):
                hv = (iota + (h0 + tt * 128 + g * 16)).astype(jnp.uint32)
                hp = hv * P2
                for l in range(NL):
                    cl = np.uint32((l * int(P1)) & 0xFFFFFFFF)
                    al = (hp ^ cl) & MASKC
                    fidx[o + tt * NL + l, pl.ds(g * 16, 16)] = (
                        al.astype(jnp.int32))
        for s in range(NROW_B):
            pltpu.async_copy(ftl_hbm.at[fidx.at[o + s]], ftb.at[o + s],
                             sem_ft)

    def load_li(b, o, sem_li):
        t0 = t_start + b * PT_B
        pltpu.async_copy(li_hbm.at[pl.ds(t0 * NL, PT_B * NL)],
                         itbuf.at[pl.ds(o, PT_B * NL)], sem_li)

    def consume(b, o, sem_li, sem_ft):
        h0 = (t_start + b * PT_B) * 128
        pltpu.make_async_copy(
            li_hbm.at[pl.ds(t_start * NL, PT_B * NL)],
            itbuf.at[pl.ds(o, PT_B * NL)], sem_li).wait()
        for tt in range(PT_B):
            for g in range(8):
                vl = [itbuf[o + tt * NL + l, pl.ds(g * 16, 16)]
                      for l in range(NL)]
                mx = vl[0]
                for l in range(1, NL):
                    mx = jnp.maximum(mx, vl[l])
                ex = [jnp.exp(v - mx) for v in vl]
                s = ex[0]
                for l in range(1, NL):
                    s = s + ex[l]
                inv = 1.0 / s
                for l in range(NL):
                    cob[tt * NL + l, pl.ds(g * 16, 16)] = ex[l] * inv
        for s in range(NROW_B):
            pltpu.make_async_copy(ftl_hbm.at[fidx.at[o + s]],
                                  ftb.at[o + s], sem_ft).wait()
        for tt in range(PT_B):
            for g in range(8):
                ii = iota + g * 16
                row = iota + (tt * 128 + g * 16)
                acc = [jnp.zeros((16,), jnp.float32) for _ in range(NF)]
                for l in range(NL):
                    cv = cob[tt * NL + l, pl.ds(g * 16, 16)]
                    for f in range(NF):
                        fv = plsc.load_gather(
                            ftb, [_splat(tt * NL + l) + o, ii, _splat(f)])
                        acc[f] = acc[f] + cv * fv
                for f in range(NF):
                    plsc.store_scatter(fbuf, [row, _splat(f)], acc[f])
        pltpu.sync_copy(fbuf, f_hbm.at[pl.ds(h0, PT_B * 128)])

    load_li(0, 0, sem_li0)
    stage(0, 0, sem_ft0)

    def two_blocks(i, carry):
        b0 = i * 2
        load_li(b0 + 1, NROW_B, sem_li1)
        stage(b0 + 1, NROW_B, sem_ft1)
        consume(b0, 0, sem_li0, sem_ft0)

        @pl.when(b0 + 2 < PNB)
        def _():
            load_li(b0 + 2, 0, sem_li0)
            stage(b0 + 2, 0, sem_ft0)

        consume(b0 + 1, NROW_B, sem_li1, sem_ft1)
        return carry

    lax.fori_loop(0, PNB // 2, two_blocks, 0)


def _final_body(lx_hbm, f_hbm, lo_hbm, xbuf, hidx, w_buf, fb, ob,
                semx0, semx1, semf0, semf1):
    cid = lax.axis_index("c")
    sid = lax.axis_index("s")
    wid = sid * NC + cid
    iota = lax.iota(jnp.int32, 16)

    def load_x(k, o, semx):
        t = wid * NCHUNK + k
        pltpu.async_copy(lx_hbm.at[pl.ds(t * 4, 4)],
                         xbuf.at[pl.ds(o * 4, 4)], semx)

    def hash_fire(k, o, semx, semf):
        pltpu.make_async_copy(lx_hbm.at[pl.ds(wid * 4, 4)],
                              xbuf.at[pl.ds(o * 4, 4)], semx).wait()

        def phase_a(g, c2):
            n0 = g * 16
            xs = [xbuf[o * 4 + d, pl.ds(n0, 16)] * RESOLUTION
                  for d in range(DIM)]
            xi = [v.astype(jnp.int32) for v in xs]
            xf = [xs[d] - xi[d].astype(jnp.float32) for d in range(DIM)]
            t0 = [1.0 - v for v in xf]
            xu = [v.astype(jnp.uint32) for v in xi]
            m0 = [xu[0], xu[1] * P1, xu[2] * P2]
            m1 = [m0[0] + np.uint32(1), m0[1] + P1, m0[2] + P2]
            for c in range(8):
                b = [(c >> d) & 1 for d in range(DIM)]
                h = ((m1[0] if b[0] else m0[0])
                     ^ (m1[1] if b[1] else m0[1])
                     ^ (m1[2] if b[2] else m0[2]))
                hidx[o * 8 + c, pl.ds(n0, 16)] = (h & MASKC).astype(jnp.int32)
                w = ((xf[0] if b[0] else t0[0])
                     * (xf[1] if b[1] else t0[1])
                     * (xf[2] if b[2] else t0[2]))
                w_buf[o * 8 + c, pl.ds(n0, 16)] = w
            return c2

        lax.fori_loop(0, NGRP, phase_a, 0)
        for j in range(8):
            pltpu.async_copy(f_hbm.at[hidx.at[o * 8 + j]],
                             fb.at[o * 8 + j], semf)

    def consume(k, o, semf):
        t = wid * NCHUNK + k
        for j in range(8):
            pltpu.make_async_copy(f_hbm.at[hidx.at[o * 8 + j]],
                                  fb.at[o * 8 + j], semf).wait()

        def reduce_g(g, c2):
            n0 = g * 16
            ii = iota + n0
            acc = [jnp.zeros((16,), jnp.float32) for _ in range(NF)]
            for c in range(8):
                wv = w_buf[o * 8 + c, pl.ds(n0, 16)]
                for f in range(NF):
                    fv = plsc.load_gather(fb, [_splat(o * 8 + c), ii,
                                               _splat(f)])
                    acc[f] = acc[f] + wv * fv
            for f in range(NF):
                ob[f, pl.ds(n0, 16)] = acc[f]
            return c2

        lax.fori_loop(0, NGRP, reduce_g, 0)
        pltpu.sync_copy(ob, lo_hbm.at[pl.ds(t * 4, 4)])

    load_x(0, 0, semx0)
    hash_fire(0, 0, semx0, semf0)

    def two_chunks(i, carry):
        k0 = i * 2
        load_x(k0 + 1, 1, semx1)
        hash_fire(k0 + 1, 1, semx1, semf1)
        consume(k0, 0, semf0)

        @pl.when(k0 + 2 < NCHUNK)
        def _():
            load_x(k0 + 2, 0, semx0)
            hash_fire(k0 + 2, 0, semx0, semf0)

        consume(k0 + 1, 1, semf1)
        return carry

    lax.fori_loop(0, NCHUNK // 2, two_chunks, 0)


@jax.jit
def _run(x, feature_table, index_table):
    mesh = plsc.VectorSubcoreMesh(core_axis_name="c", subcore_axis_name="s")
    params = pltpu.CompilerParams(
        needs_layout_passes=False, use_tc_tiling_on_sc=False)

    detile_ft = pl.kernel(
        _detile_ft_body,
        mesh=mesh,
        compiler_params=params,
        out_type=jax.ShapeDtypeStruct((TBL, NL), jnp.float32),
        scratch_types=[
            pltpu.VMEM((TPB * NF, 128), jnp.float32),
            pltpu.VMEM((TPB * 128, NL), jnp.float32),
            pltpu.SemaphoreType.DMA,
        ],
    )

    precompute = pl.kernel(
        _precompute_body,
        mesh=mesh,
        compiler_params=params,
        out_type=jax.ShapeDtypeStruct((TBL, NL), jnp.float32),
        scratch_types=[
            pltpu.VMEM((2 * NROW_B, 128), jnp.float32),
            pltpu.VMEM((2 * NROW_B, 128), jnp.int32),
            pltpu.VMEM((2 * NROW_B, 128, NL), jnp.float32),
            pltpu.VMEM((NROW_B, 128), jnp.float32),
            pltpu.VMEM((PT_B * 128, NL), jnp.float32),
            pltpu.SemaphoreType.DMA,
            pltpu.SemaphoreType.DMA,
            pltpu.SemaphoreType.DMA,
            pltpu.SemaphoreType.DMA,
        ],
    )

    final = pl.kernel(
        _final_body,
        mesh=mesh,
        compiler_params=params,
        out_type=jax.ShapeDtypeStruct((NPTS // 128 * 4, 128), jnp.float32),
        scratch_types=[
            pltpu.VMEM((2 * 4, 128), jnp.float32),
            pltpu.VMEM((2 * 8, CHK), jnp.int32),
            pltpu.VMEM((2 * 8, CHK), jnp.float32),
            pltpu.VMEM((2 * 8, CHK, NL), jnp.float32),
            pltpu.VMEM((4, 128), jnp.float32),
            pltpu.SemaphoreType.DMA,
            pltpu.SemaphoreType.DMA,
            pltpu.SemaphoreType.DMA,
            pltpu.SemaphoreType.DMA,
        ],
    )

    li = index_table.T.reshape(NL, NTILE_T, 128).transpose(1, 0, 2)
    li = li.reshape(NTILE_T * NL, 128)
    lf = feature_table.T.reshape(NF, NTILE_T, 128).transpose(1, 0, 2)
    lf = lf.reshape(NTILE_T * NF, 128)
    xp = jnp.pad(x, ((0, 0), (0, 1)))
    lx = xp.T.reshape(4, NPTS // 128, 128).transpose(1, 0, 2)
    lx = lx.reshape(NPTS // 128 * 4, 128)

    ftl = detile_ft(lf)
    fq = precompute(li, ftl)
    lo = final(lx, fq)

    out = lo.reshape(NPTS // 128, 4, 128).transpose(1, 0, 2)
    return out.reshape(4, NPTS).T


def kernel(x, feature_table, index_table):
    return _run(x, feature_table, index_table)
